# baseline (speedup 1.0000x reference)
import functools

import jax
import jax.numpy as jnp
from jax import lax
from jax.experimental import pallas as pl
from jax.experimental.pallas import tpu as pltpu
from jax.experimental.pallas import tpu_sc as plsc

N = 10000
NFEAT = 128
NHID = 256
NCLASS = 40
NCLS_PAD = 64
EPS = 1e-5

NS = 16
GROUP = 128
ACC_ROWS = 10240
ZR = ACC_ROWS // NS
ZR_MAIN = N - (NS - 1) * ZR
CHUNK = 32


def _make_sc_aggregate(d: int, g: int):
    mesh = plsc.VectorSubcoreMesh(core_axis_name="c", subcore_axis_name="s")

    @functools.partial(
        pl.kernel,
        out_type=jax.ShapeDtypeStruct((N, d), jnp.float32),
        mesh=mesh,
        compiler_params=pltpu.CompilerParams(use_tc_tiling_on_sc=False),
        scratch_types=[
            pltpu.VMEM((CHUNK, GROUP), jnp.int32),
            pltpu.VMEM((CHUNK, GROUP), jnp.int32),
            pltpu.VMEM((GROUP, d), jnp.float32),
            pltpu.VMEM((GROUP, d), jnp.float32),
            pltpu.VMEM_SHARED((ACC_ROWS, d), jnp.float32),
            pltpu.SemaphoreType.DMA,
            pltpu.SemaphoreType.DMA,
        ],
    )
    def agg(table, src_r, dst_r, out, src_v, dst_v, rows_a, rows_b,
            acc, sem_a, sem_b):
        c = lax.axis_index("c")
        s = lax.axis_index("s")

        @pl.when(c == 0)
        def _all():
            def zbody(i, carry):
                for k in range(d // 16):
                    rows_a[i, pl.ds(16 * k, 16)] = jnp.zeros((16,),
                                                             jnp.float32)
                return carry

            lax.fori_loop(0, GROUP, zbody, 0)
            for r in range(ZR // GROUP):
                pltpu.sync_copy(rows_a,
                                acc.at[pl.ds(s * ZR + r * GROUP, GROUP)])
            plsc.subcore_barrier()

            gbase = s * g
            for off in range(0, g, CHUNK):
                cs = min(CHUNK, g - off)
                pltpu.sync_copy(src_r.at[pl.ds(gbase + off, cs)],
                                src_v.at[pl.ds(0, cs)])
                pltpu.sync_copy(dst_r.at[pl.ds(gbase + off, cs)],
                                dst_v.at[pl.ds(0, cs)])
                pltpu.async_copy(table.at[src_v.at[0]], rows_a, sem_a)

                def body(j2, carry, cs=cs):
                    j = j2 * 2
                    g_b = pltpu.async_copy(table.at[src_v.at[j + 1]], rows_b,
                                           sem_b)
                    pltpu.make_async_copy(table.at[src_v.at[0]], rows_a,
                                          sem_a).wait()
                    pltpu.sync_copy(rows_a, acc.at[dst_v.at[j]], add=True)

                    @pl.when(j + 2 < cs)
                    def _():
                        pltpu.async_copy(table.at[src_v.at[j + 2]], rows_a,
                                         sem_a)

                    g_b.wait()
                    pltpu.sync_copy(rows_b, acc.at[dst_v.at[j + 1]],
                                    add=True)
                    return carry

                lax.fori_loop(0, cs // 2, body, 0)

            plsc.subcore_barrier()

            base = s * ZR
            pltpu.sync_copy(acc.at[pl.ds(base, ZR_MAIN)],
                            out.at[pl.ds(base, ZR_MAIN)])

            @pl.when(s < NS - 1)
            def _():
                pltpu.sync_copy(acc.at[pl.ds(base + ZR_MAIN, ZR - ZR_MAIN)],
                                out.at[pl.ds(base + ZR_MAIN, ZR - ZR_MAIN)])

    return agg


_BM = 400


def _tc1_body(a0, w1, b1, h_ref, sums_ref):
    h = jnp.dot(a0[...], w1[...], preferred_element_type=jnp.float32) + b1[...]
    h_ref[...] = h

    @pl.when(pl.program_id(0) == 0)
    def _():
        sums_ref[...] = jnp.zeros_like(sums_ref)

    sums_ref[0:1, :] += jnp.sum(h, axis=0, keepdims=True)
    sums_ref[1:2, :] += jnp.sum(h * h, axis=0, keepdims=True)


def _tc2_body(h_ref, sums, gamma, beta, w2, y_ref):
    mean = sums[0:1, :] * (1.0 / N)
    var = sums[1:2, :] * (1.0 / N) - mean * mean
    inv = lax.rsqrt(var + EPS)
    hn = (h_ref[...] - mean) * (inv * gamma[...]) + beta[...]
    hr = jnp.maximum(hn, 0.0)
    y_ref[...] = jnp.dot(hr, w2[...], preferred_element_type=jnp.float32)


def _tc3_body(p0, b2, out_ref):
    out_ref[...] = p0[...][:, :NCLASS] + b2[...]


def kernel(x, edge_index, W1, b1, gamma, beta, W2, b2):
    e = edge_index.shape[1]
    g = -(-(-(-e // GROUP)) // NS)
    g = -(-g // 8) * 8
    g_total = NS * g
    e_pad = g_total * GROUP

    src = jnp.concatenate(
        [edge_index[0], jnp.zeros((e_pad - e,), jnp.int32)]).reshape(
            g_total, GROUP)
    dummy_dst = N + jnp.arange(e_pad - e, dtype=jnp.int32) % (ACC_ROWS - N)
    dst = jnp.concatenate([edge_index[1], dummy_dst]).reshape(
        g_total, GROUP)

    w2p = jnp.pad(W2, ((0, 0), (0, NCLS_PAD - NCLASS)))

    agg1 = _make_sc_aggregate(NFEAT, g)(x, src, dst)

    grid = (N // _BM,)
    h, sums = pl.pallas_call(
        _tc1_body,
        grid=grid,
        in_specs=[
            pl.BlockSpec((_BM, NFEAT), lambda i: (i, 0)),
            pl.BlockSpec((NFEAT, NHID), lambda i: (0, 0)),
            pl.BlockSpec((1, NHID), lambda i: (0, 0)),
        ],
        out_specs=[
            pl.BlockSpec((_BM, NHID), lambda i: (i, 0)),
            pl.BlockSpec((2, NHID), lambda i: (0, 0)),
        ],
        out_shape=[
            jax.ShapeDtypeStruct((N, NHID), jnp.float32),
            jax.ShapeDtypeStruct((2, NHID), jnp.float32),
        ],
    )(agg1, W1, b1.reshape(1, NHID))

    y = pl.pallas_call(
        _tc2_body,
        grid=grid,
        in_specs=[
            pl.BlockSpec((_BM, NHID), lambda i: (i, 0)),
            pl.BlockSpec((2, NHID), lambda i: (0, 0)),
            pl.BlockSpec((1, NHID), lambda i: (0, 0)),
            pl.BlockSpec((1, NHID), lambda i: (0, 0)),
            pl.BlockSpec((NHID, NCLS_PAD), lambda i: (0, 0)),
        ],
        out_specs=pl.BlockSpec((_BM, NCLS_PAD), lambda i: (i, 0)),
        out_shape=jax.ShapeDtypeStruct((N, NCLS_PAD), jnp.float32),
    )(h, sums, gamma.reshape(1, NHID), beta.reshape(1, NHID), w2p)

    agg2 = _make_sc_aggregate(NCLS_PAD, g)(y, src, dst)

    out = pl.pallas_call(
        _tc3_body,
        grid=grid,
        in_specs=[
            pl.BlockSpec((_BM, NCLS_PAD), lambda i: (i, 0)),
            pl.BlockSpec((1, NCLASS), lambda i: (0, 0)),
        ],
        out_specs=pl.BlockSpec((_BM, NCLASS), lambda i: (i, 0)),
        out_shape=jax.ShapeDtypeStruct((N, NCLASS), jnp.float32),
    )(agg2, b2.reshape(1, NCLASS))

    return out


# speedup vs baseline: 2.9262x; 2.9262x over previous
import functools

import jax
import jax.numpy as jnp
from jax import lax
from jax.experimental import pallas as pl
from jax.experimental.pallas import tpu as pltpu
from jax.experimental.pallas import tpu_sc as plsc

N = 10000
NFEAT = 128
NHID = 256
NCLASS = 40
NCLS_PAD = 64
EPS = 1e-5

NC = 2
NS = 16
GROUP = 128
ACC_ROWS = 10240
ZR = ACC_ROWS // NS
ZR_MAIN = N - (NS - 1) * ZR
CHUNK = 32


def _make_sc_aggregate(d: int, g: int):
    mesh = plsc.VectorSubcoreMesh(core_axis_name="c", subcore_axis_name="s")

    @functools.partial(
        pl.kernel,
        out_type=jax.ShapeDtypeStruct((NC, N, d), jnp.float32),
        mesh=mesh,
        compiler_params=pltpu.CompilerParams(use_tc_tiling_on_sc=False),
        scratch_types=[
            pltpu.VMEM((CHUNK, GROUP), jnp.int32),
            pltpu.VMEM((CHUNK, GROUP), jnp.int32),
            pltpu.VMEM((GROUP, d), jnp.float32),
            pltpu.VMEM((GROUP, d), jnp.float32),
            pltpu.VMEM_SHARED((ACC_ROWS, d), jnp.float32),
            pltpu.SemaphoreType.DMA,
            pltpu.SemaphoreType.DMA,
        ],
    )
    def agg(table, src_r, dst_r, out, src_v, dst_v, rows_a, rows_b,
            acc, sem_a, sem_b):
        c = lax.axis_index("c")
        s = lax.axis_index("s")

        def zbody(i, carry):
            for k in range(d // 16):
                rows_a[i, pl.ds(16 * k, 16)] = jnp.zeros((16,), jnp.float32)
            return carry

        lax.fori_loop(0, GROUP, zbody, 0)
        for r in range(ZR // GROUP):
            pltpu.sync_copy(rows_a, acc.at[pl.ds(s * ZR + r * GROUP, GROUP)])
        plsc.subcore_barrier()

        gbase = (s * NC + c) * g
        for off in range(0, g, CHUNK):
            cs = min(CHUNK, g - off)
            pltpu.sync_copy(src_r.at[pl.ds(gbase + off, cs)],
                            src_v.at[pl.ds(0, cs)])
            pltpu.sync_copy(dst_r.at[pl.ds(gbase + off, cs)],
                            dst_v.at[pl.ds(0, cs)])
            pltpu.async_copy(table.at[src_v.at[0]], rows_a, sem_a)

            def body(j2, carry, cs=cs):
                j = j2 * 2
                g_b = pltpu.async_copy(table.at[src_v.at[j + 1]], rows_b,
                                       sem_b)
                pltpu.make_async_copy(table.at[src_v.at[0]], rows_a,
                                      sem_a).wait()
                pltpu.sync_copy(rows_a, acc.at[dst_v.at[j]], add=True)

                @pl.when(j + 2 < cs)
                def _():
                    pltpu.async_copy(table.at[src_v.at[j + 2]], rows_a,
                                     sem_a)

                g_b.wait()
                pltpu.sync_copy(rows_b, acc.at[dst_v.at[j + 1]], add=True)
                return carry

            lax.fori_loop(0, cs // 2, body, 0)

        plsc.subcore_barrier()

        base = s * ZR
        pltpu.sync_copy(acc.at[pl.ds(base, ZR_MAIN)],
                        out.at[c].at[pl.ds(base, ZR_MAIN)])

        @pl.when(s < NS - 1)
        def _():
            pltpu.sync_copy(acc.at[pl.ds(base + ZR_MAIN, ZR - ZR_MAIN)],
                            out.at[c].at[pl.ds(base + ZR_MAIN, ZR - ZR_MAIN)])

    return agg


_BM = 400


def _tc1_body(a0, a1, w1, b1, h_ref, sums_ref):
    a = a0[...] + a1[...]
    h = jnp.dot(a, w1[...], preferred_element_type=jnp.float32) + b1[...]
    h_ref[...] = h

    @pl.when(pl.program_id(0) == 0)
    def _():
        sums_ref[...] = jnp.zeros_like(sums_ref)

    sums_ref[0:1, :] += jnp.sum(h, axis=0, keepdims=True)
    sums_ref[1:2, :] += jnp.sum(h * h, axis=0, keepdims=True)


def _tc2_body(h_ref, sums, gamma, beta, w2, y_ref):
    mean = sums[0:1, :] * (1.0 / N)
    var = sums[1:2, :] * (1.0 / N) - mean * mean
    inv = lax.rsqrt(var + EPS)
    hn = (h_ref[...] - mean) * (inv * gamma[...]) + beta[...]
    hr = jnp.maximum(hn, 0.0)
    y_ref[...] = jnp.dot(hr, w2[...], preferred_element_type=jnp.float32)


def _tc3_body(p0, p1, b2, out_ref):
    t = p0[...] + p1[...]
    out_ref[...] = t[:, :NCLASS] + b2[...]


def kernel(x, edge_index, W1, b1, gamma, beta, W2, b2):
    e = edge_index.shape[1]
    g = -(-(-(-e // GROUP)) // (NC * NS))
    g = -(-g // 8) * 8
    g_total = NC * NS * g
    e_pad = g_total * GROUP

    pad_idx = jnp.arange(e_pad - e, dtype=jnp.int32)
    dummy_src = pad_idx % N
    dummy_dst = N + pad_idx % (ACC_ROWS - N)
    src = jnp.concatenate([edge_index[0], dummy_src]).reshape(g_total, GROUP)
    dst = jnp.concatenate([edge_index[1], dummy_dst]).reshape(g_total, GROUP)

    w2p = jnp.pad(W2, ((0, 0), (0, NCLS_PAD - NCLASS)))

    agg1 = _make_sc_aggregate(NFEAT, g)(x, src, dst)

    grid = (N // _BM,)
    h, sums = pl.pallas_call(
        _tc1_body,
        grid=grid,
        in_specs=[
            pl.BlockSpec((_BM, NFEAT), lambda i: (i, 0)),
            pl.BlockSpec((_BM, NFEAT), lambda i: (i, 0)),
            pl.BlockSpec((NFEAT, NHID), lambda i: (0, 0)),
            pl.BlockSpec((1, NHID), lambda i: (0, 0)),
        ],
        out_specs=[
            pl.BlockSpec((_BM, NHID), lambda i: (i, 0)),
            pl.BlockSpec((2, NHID), lambda i: (0, 0)),
        ],
        out_shape=[
            jax.ShapeDtypeStruct((N, NHID), jnp.float32),
            jax.ShapeDtypeStruct((2, NHID), jnp.float32),
        ],
    )(agg1[0], agg1[1], W1, b1.reshape(1, NHID))

    y = pl.pallas_call(
        _tc2_body,
        grid=grid,
        in_specs=[
            pl.BlockSpec((_BM, NHID), lambda i: (i, 0)),
            pl.BlockSpec((2, NHID), lambda i: (0, 0)),
            pl.BlockSpec((1, NHID), lambda i: (0, 0)),
            pl.BlockSpec((1, NHID), lambda i: (0, 0)),
            pl.BlockSpec((NHID, NCLS_PAD), lambda i: (0, 0)),
        ],
        out_specs=pl.BlockSpec((_BM, NCLS_PAD), lambda i: (i, 0)),
        out_shape=jax.ShapeDtypeStruct((N, NCLS_PAD), jnp.float32),
    )(h, sums, gamma.reshape(1, NHID), beta.reshape(1, NHID), w2p)

    agg2 = _make_sc_aggregate(NCLS_PAD, g)(y, src, dst)

    out = pl.pallas_call(
        _tc3_body,
        grid=grid,
        in_specs=[
            pl.BlockSpec((_BM, NCLS_PAD), lambda i: (i, 0)),
            pl.BlockSpec((_BM, NCLS_PAD), lambda i: (i, 0)),
            pl.BlockSpec((1, NCLASS), lambda i: (0, 0)),
        ],
        out_specs=pl.BlockSpec((_BM, NCLASS), lambda i: (i, 0)),
        out_shape=jax.ShapeDtypeStruct((N, NCLASS), jnp.float32),
    )(agg2[0], agg2[1], b2.reshape(1, NCLASS))

    return out
